# baseline (speedup 1.0000x reference)
import functools

import jax
import jax.numpy as jnp
from jax.experimental import pallas as pl
from jax.experimental.pallas import tpu as pltpu

_L = 21
_TOT = 128
_RBITS = 8192


def _body(roi_ref, g_ref, rp_ref, rn_ref, olab_ref, od_ref, *, n, m):
    f32 = jnp.float32
    roi = jnp.transpose(roi_ref[0], (1, 0))
    by1, bx1, by2, bx2 = (roi[i : i + 1, :] for i in range(4))
    g = g_ref[0]
    gy1, gx1, gy2, gx2 = (g[:, i : i + 1] for i in range(4))

    barea = (by2 - by1) * (bx2 - bx1)
    garea = (gy2 - gy1) * (gx2 - gx1)
    xt = jnp.maximum(bx1, gx1)
    yt = jnp.maximum(by1, gy1)
    xb = jnp.minimum(bx2, gx2)
    yb = jnp.minimum(by2, gy2)
    inter = jnp.maximum(xb - xt, 0.0) * jnp.maximum(yb - yt, 0.0)
    union = barea + garea - inter
    iou = inter / union

    merged = jnp.max(iou, axis=0, keepdims=True)
    iota_m = jax.lax.broadcasted_iota(jnp.int32, (m, 1), 0)
    midx = jnp.min(jnp.where(iou == merged, iota_m, m), axis=0, keepdims=True)
    onehot = iota_m == midx
    def gather_col(c):
        return jnp.sum(
            jnp.where(onehot, g[:, c : c + 1], 0.0), axis=0, keepdims=True
        )

    ggy1, ggx1, ggy2, ggx2 = (gather_col(c) for c in range(4))
    glab = gather_col(4).astype(jnp.int32)

    pos0 = merged > 0.5
    neg0 = jnp.logical_and(merged < 0.5, merged > 0.1)
    iota_n = jax.lax.broadcasted_iota(jnp.int32, (1, n), 1)
    kp = jnp.where(pos0, rp_ref[0], 0) * _RBITS + (_RBITS - 1 - iota_n)
    kn = jnp.where(neg0, rn_ref[0], 0) * _RBITS + (_RBITS - 1 - iota_n)
    kb = jnp.concatenate([kp, kn], axis=0)

    def bs_body(_, c):
        lo, hi = c
        mid = lo + (hi - lo) // 2
        cnt = jnp.sum((kb > mid).astype(jnp.int32), axis=1, keepdims=True)
        ge = cnt >= _TOT
        return jnp.where(ge, mid, lo), jnp.where(ge, hi, mid)

    lo0 = jnp.full((2, 1), -1, jnp.int32)
    hi0 = jnp.full((2, 1), 1 << 24, jnp.int32)
    lo, _ = jax.lax.fori_loop(0, 26, bs_body, (lo0, hi0))
    sel = kb > lo
    pos = jnp.logical_and(sel[0:1, :], pos0)
    neg = jnp.logical_and(sel[1:2, :], neg0)

    lab = jnp.where(pos, glab, jnp.where(neg, 0, -1))

    zero = jnp.zeros_like(merged)
    ey1 = jnp.where(pos, ggy1, zero)
    ex1 = jnp.where(pos, ggx1, zero)
    ey2 = jnp.where(pos, ggy2, zero)
    ex2 = jnp.where(pos, ggx2, zero)
    bw = bx2 - bx1
    bh = by2 - by1
    bcx = bx1 + 0.5 * bw
    bcy = by1 + 0.5 * bh
    gw = ex2 - ex1
    gh = ey2 - ey1
    gcx = ex1 + 0.5 * gw
    gcy = ey1 + 0.5 * gh
    bw = jnp.where(bw == 0, 1e-3, bw)
    bh = jnp.where(bh == 0, 1e-3, bh)
    gws = jnp.where(gw == 0, 1.0, gw)
    ghs = jnp.where(gh == 0, 1.0, gh)
    dx = jnp.where(gw == 0, 0.0, (gcx - bcx) / bw) / 0.1
    dy = jnp.where(gh == 0, 0.0, (gcy - bcy) / bh) / 0.1
    dw = jnp.where(gw == 0, 0.0, jnp.log(gws / bw)) / 0.2
    dh = jnp.where(gh == 0, 0.0, jnp.log(ghs / bh)) / 0.2

    io21 = jax.lax.broadcasted_iota(jnp.int32, (_L, 1), 0)
    olabT = (io21 == lab).astype(f32)
    io84 = jax.lax.broadcasted_iota(jnp.int32, (4 * _L, 1), 0)
    oh84 = (io84 // 4 == lab).astype(f32)
    d4 = jnp.concatenate([dy, dx, dh, dw], axis=0)
    dtile = jnp.concatenate([d4] * _L, axis=0)
    od84T = oh84 * dtile

    olab_ref[0] = jnp.transpose(olabT, (1, 0))
    od_ref[0] = jnp.transpose(od84T, (1, 0))


@jax.jit
def kernel(roi_bboxes, gt_boxes, gt_labels):
    f32 = jnp.float32
    B, N, _ = roi_bboxes.shape
    M = gt_boxes.shape[1]

    rand_pos = jax.random.randint(jax.random.key(11), (B, N), 1, _TOT * 10)
    rand_neg = jax.random.randint(jax.random.key(13), (B, N), 1, _TOT * 10)

    g_aug = jnp.concatenate(
        [gt_boxes, gt_labels[..., None].astype(f32), jnp.zeros((B, M, 3), f32)],
        axis=-1,
    )

    out_lab, out_d84 = pl.pallas_call(
        functools.partial(_body, n=N, m=M),
        grid=(B,),
        in_specs=[
            pl.BlockSpec((1, N, 4), lambda b: (b, 0, 0)),
            pl.BlockSpec((1, M, 8), lambda b: (b, 0, 0)),
            pl.BlockSpec((1, 1, N), lambda b: (b, 0, 0)),
            pl.BlockSpec((1, 1, N), lambda b: (b, 0, 0)),
        ],
        out_specs=[
            pl.BlockSpec((1, N, _L), lambda b: (b, 0, 0)),
            pl.BlockSpec((1, N, 4 * _L), lambda b: (b, 0, 0)),
        ],
        out_shape=[
            jax.ShapeDtypeStruct((B, N, _L), f32),
            jax.ShapeDtypeStruct((B, N, 4 * _L), f32),
        ],
        compiler_params=pltpu.CompilerParams(
            dimension_semantics=("parallel",)
        ),
    )(roi_bboxes, g_aug, rand_pos.reshape(B, 1, N), rand_neg.reshape(B, 1, N))

    return out_d84.reshape(B, N, _L, 4), out_lab


# speedup vs baseline: 1.1709x; 1.1709x over previous
import functools

import jax
import jax.numpy as jnp
from jax.experimental import pallas as pl
from jax.experimental.pallas import tpu as pltpu

_L = 21
_TOT = 128
_RBITS = 8192


def _body(roi_ref, g_ref, rp_ref, rn_ref, olab_ref, od_ref, *, n, m):
    f32 = jnp.float32
    roi = roi_ref[0]
    by1, bx1, by2, bx2 = (roi[i : i + 1, :] for i in range(4))
    g = g_ref[0]
    gy1, gx1, gy2, gx2 = (g[:, i : i + 1] for i in range(4))

    barea = (by2 - by1) * (bx2 - bx1)
    garea = (gy2 - gy1) * (gx2 - gx1)
    xt = jnp.maximum(bx1, gx1)
    yt = jnp.maximum(by1, gy1)
    xb = jnp.minimum(bx2, gx2)
    yb = jnp.minimum(by2, gy2)
    inter = jnp.maximum(xb - xt, 0.0) * jnp.maximum(yb - yt, 0.0)
    union = barea + garea - inter
    iou = inter / union

    merged = jnp.max(iou, axis=0, keepdims=True)
    iota_m = jax.lax.broadcasted_iota(jnp.int32, (m, 1), 0)
    midx = jnp.min(jnp.where(iou == merged, iota_m, m), axis=0, keepdims=True)
    onehot = iota_m == midx
    def gather_col(c):
        return jnp.sum(
            jnp.where(onehot, g[:, c : c + 1], 0.0), axis=0, keepdims=True
        )

    ggy1, ggx1, ggy2, ggx2 = (gather_col(c) for c in range(4))
    glab = gather_col(4).astype(jnp.int32)

    pos0 = merged > 0.5
    neg0 = jnp.logical_and(merged < 0.5, merged > 0.1)
    iota_n = jax.lax.broadcasted_iota(jnp.int32, (1, n), 1)
    kp = jnp.where(pos0, rp_ref[0], 0) * _RBITS + (_RBITS - 1 - iota_n)
    kn = jnp.where(neg0, rn_ref[0], 0) * _RBITS + (_RBITS - 1 - iota_n)
    kb = jnp.concatenate([kp, kn], axis=0)

    def bs_body(_, c):
        lo, hi = c
        mid = lo + (hi - lo) // 2
        cnt = jnp.sum((kb > mid).astype(jnp.int32), axis=1, keepdims=True)
        ge = cnt >= _TOT
        return jnp.where(ge, mid, lo), jnp.where(ge, hi, mid)

    lo0 = jnp.full((2, 1), -1, jnp.int32)
    hi0 = jnp.full((2, 1), 1 << 24, jnp.int32)
    lo, _ = jax.lax.fori_loop(0, 26, bs_body, (lo0, hi0))
    sel = kb > lo
    pos = jnp.logical_and(sel[0:1, :], pos0)
    neg = jnp.logical_and(sel[1:2, :], neg0)

    lab = jnp.where(pos, glab, jnp.where(neg, 0, -1))

    zero = jnp.zeros_like(merged)
    ey1 = jnp.where(pos, ggy1, zero)
    ex1 = jnp.where(pos, ggx1, zero)
    ey2 = jnp.where(pos, ggy2, zero)
    ex2 = jnp.where(pos, ggx2, zero)
    bw = bx2 - bx1
    bh = by2 - by1
    bcx = bx1 + 0.5 * bw
    bcy = by1 + 0.5 * bh
    gw = ex2 - ex1
    gh = ey2 - ey1
    gcx = ex1 + 0.5 * gw
    gcy = ey1 + 0.5 * gh
    bw = jnp.where(bw == 0, 1e-3, bw)
    bh = jnp.where(bh == 0, 1e-3, bh)
    gws = jnp.where(gw == 0, 1.0, gw)
    ghs = jnp.where(gh == 0, 1.0, gh)
    dx = jnp.where(gw == 0, 0.0, (gcx - bcx) / bw) / 0.1
    dy = jnp.where(gh == 0, 0.0, (gcy - bcy) / bh) / 0.1
    dw = jnp.where(gw == 0, 0.0, jnp.log(gws / bw)) / 0.2
    dh = jnp.where(gh == 0, 0.0, jnp.log(ghs / bh)) / 0.2

    io21 = jax.lax.broadcasted_iota(jnp.int32, (_L, 1), 0)
    olabT = (io21 == lab).astype(f32)
    io84 = jax.lax.broadcasted_iota(jnp.int32, (4 * _L, 1), 0)
    oh84 = (io84 // 4 == lab).astype(f32)
    d4 = jnp.concatenate([dy, dx, dh, dw], axis=0)
    dtile = jnp.concatenate([d4] * _L, axis=0)
    od84T = oh84 * dtile

    olab_ref[0] = olabT
    od_ref[0] = od84T


@jax.jit
def kernel(roi_bboxes, gt_boxes, gt_labels):
    f32 = jnp.float32
    B, N, _ = roi_bboxes.shape
    M = gt_boxes.shape[1]

    rand_pos = jax.random.randint(jax.random.key(11), (B, N), 1, _TOT * 10)
    rand_neg = jax.random.randint(jax.random.key(13), (B, N), 1, _TOT * 10)

    roi_t = jnp.transpose(roi_bboxes, (0, 2, 1))
    g_aug = jnp.concatenate(
        [gt_boxes, gt_labels[..., None].astype(f32), jnp.zeros((B, M, 3), f32)],
        axis=-1,
    )

    out_lab, out_d84 = pl.pallas_call(
        functools.partial(_body, n=N, m=M),
        grid=(B,),
        in_specs=[
            pl.BlockSpec((1, 4, N), lambda b: (b, 0, 0)),
            pl.BlockSpec((1, M, 8), lambda b: (b, 0, 0)),
            pl.BlockSpec((1, 1, N), lambda b: (b, 0, 0)),
            pl.BlockSpec((1, 1, N), lambda b: (b, 0, 0)),
        ],
        out_specs=[
            pl.BlockSpec((1, _L, N), lambda b: (b, 0, 0)),
            pl.BlockSpec((1, 4 * _L, N), lambda b: (b, 0, 0)),
        ],
        out_shape=[
            jax.ShapeDtypeStruct((B, _L, N), f32),
            jax.ShapeDtypeStruct((B, 4 * _L, N), f32),
        ],
        compiler_params=pltpu.CompilerParams(
            dimension_semantics=("parallel",)
        ),
    )(roi_t, g_aug, rand_pos.reshape(B, 1, N), rand_neg.reshape(B, 1, N))

    out_d = jnp.transpose(out_d84, (0, 2, 1)).reshape(B, N, _L, 4)
    return out_d, jnp.transpose(out_lab, (0, 2, 1))
